# baseline (speedup 1.0000x reference)
import jax
import jax.numpy as jnp
from jax.experimental import pallas as pl
from jax.experimental.pallas import tpu as pltpu

N_CLASSES = 1000
N_EMB = 64
BATCH = 16384
BCHUNK = 8192
NSTEPS = BATCH // BCHUNK


def _body(nbt_ref, x_ref, y_ref, centers_ref, out_ref, acc_ref):
    step = pl.program_id(0)

    @pl.when(step == 0)
    def _():
        acc_ref[...] = jnp.zeros_like(acc_ref)

    x = x_ref[...]
    y = y_ref[...]
    ids = jax.lax.broadcasted_iota(jnp.int32, (N_CLASSES, BCHUNK), 0)
    onehot_t = (y == ids).astype(jnp.bfloat16)
    xh = x.astype(jnp.bfloat16)
    xl = (x - xh.astype(jnp.float32)).astype(jnp.bfloat16)
    xa = jnp.concatenate(
        [xh, xl, jnp.ones((BCHUNK, 1), jnp.bfloat16),
         jnp.zeros((BCHUNK, 63), jnp.bfloat16)], axis=1)
    acc_ref[...] += jax.lax.dot_general(
        onehot_t, xa, (((1,), (0,)), ((), ())),
        preferred_element_type=jnp.float32)

    @pl.when(step == NSTEPS - 1)
    def _():
        acc = acc_ref[...]
        s = acc[:, :N_EMB] + acc[:, N_EMB:2 * N_EMB]
        cnt = acc[:, 2 * N_EMB:2 * N_EMB + 1]
        present = cnt > 0.0
        denom = jnp.where(present, cnt, 1.0)
        mu = s / denom
        nbt = nbt_ref[0]
        cen = centers_ref[...]
        out_ref[...] = jnp.where(present, (mu + cen * nbt) / (nbt + 1.0), cen)


_seg_update = pl.pallas_call(
    _body,
    grid=(NSTEPS,),
    out_shape=jax.ShapeDtypeStruct((N_CLASSES, N_EMB), jnp.float32),
    in_specs=[
        pl.BlockSpec(memory_space=pltpu.SMEM),
        pl.BlockSpec((BCHUNK, N_EMB), lambda i: (i, 0)),
        pl.BlockSpec((1, BCHUNK), lambda i: (0, i)),
        pl.BlockSpec((N_CLASSES, N_EMB), lambda i: (0, 0)),
    ],
    out_specs=pl.BlockSpec((N_CLASSES, N_EMB), lambda i: (0, 0)),
    scratch_shapes=[pltpu.VMEM((N_CLASSES, 192), jnp.float32)],
)


def kernel(x, y, centers, num_batches_tracked):
    new_centers = _seg_update(num_batches_tracked, x, y.reshape(1, BATCH),
                              centers)
    return (x, new_centers)
